# baseline (speedup 1.0000x reference)
import functools

import jax
import jax.numpy as jnp
import numpy as np
from jax import lax
from jax.experimental import pallas as pl
from jax.experimental.pallas import tpu as pltpu
from jax.experimental.pallas import tpu_sc as plsc

H = 128
N_NODES = 10000
N_EDGES = 320000

NC = 2
NS = 16
NW = NC * NS

EDGES_PER_TILE = N_EDGES // NW
BLK = 80
NBLK = EDGES_PER_TILE // BLK
N_ACC = 10240
ROWS_PER_TILE = N_ACC // NS
CROWS = N_ACC // H
ZCH = ROWS_PER_TILE // CROWS

MESH = plsc.VectorSubcoreMesh(core_axis_name="c", subcore_axis_name="s")


def _sc_fused(recv3d, e):
    @functools.partial(
        pl.kernel,
        out_type=(
            jax.ShapeDtypeStruct((NC * N_ACC, H), jnp.float32),
            jax.ShapeDtypeStruct((NC * CROWS, H), jnp.float32),
        ),
        mesh=MESH,
        compiler_params=pltpu.CompilerParams(needs_layout_passes=False),
        scratch_types=[
            pltpu.VMEM((NBLK, BLK), jnp.int32),
            pltpu.VMEM((BLK, H), jnp.float32),
            pltpu.VMEM((BLK, H), jnp.float32),
            pltpu.VMEM((CROWS, H), jnp.float32),
            pltpu.VMEM((CROWS,), jnp.int32),
            pltpu.VMEM_SHARED((N_ACC, H), jnp.float32),
            pltpu.VMEM_SHARED((CROWS, H), jnp.float32),
            pltpu.SemaphoreType.DMA,
            pltpu.SemaphoreType.DMA,
            pltpu.SemaphoreType.DMA,
            pltpu.SemaphoreType.DMA,
        ],
    )
    def k(recv_hbm, e_hbm,
          sums_out, cnt_out,
          idx_v, ebuf0, ebuf1, cnt2d, iota_v, acc, acc_c,
          sem0, sem1, sems0, sems1):
        c = lax.axis_index("c")
        s = lax.axis_index("s")
        w = c * NS + s
        r0 = s * ROWS_PER_TILE
        ebase = w * EDGES_PER_TILE

        pltpu.async_copy(recv_hbm.at[w], idx_v, sem1)

        z16 = jnp.zeros((16,), jnp.float32)
        for kk in range(CROWS // 16):
            iota_v[pl.ds(kk * 16, 16)] = lax.iota(jnp.int32, 16) + 16 * kk

        def zrow(j, carry):
            for kk in range(H // 16):
                cnt2d[j, pl.ds(kk * 16, 16)] = z16
            return carry

        lax.fori_loop(0, CROWS, zrow, 0)

        for zz in range(ZCH):
            pltpu.sync_copy(cnt2d, acc.at[pl.ds(r0 + zz * CROWS, CROWS)])

        @pl.when(s == 0)
        def _():
            pltpu.sync_copy(cnt2d, acc_c)

        pltpu.make_async_copy(recv_hbm.at[w], idx_v, sem1).wait()
        plsc.subcore_barrier()

        ones16 = jnp.ones((16,), jnp.float32)

        def count_block(j):
            for kk in range(BLK // 16):
                idx16 = idx_v[j, pl.ds(kk * 16, 16)]
                row = lax.shift_right_logical(idx16, 7)
                col = lax.bitwise_and(idx16, 127)
                plsc.addupdate_scatter(cnt2d, [row, col], ones16)

        def load_e(j, buf, sem):
            pltpu.async_copy(e_hbm.at[pl.ds(ebase + j * BLK, BLK)], buf, sem)

        def wait_e(buf, sem):
            pltpu.make_async_copy(e_hbm.at[pl.ds(ebase, BLK)], buf, sem).wait()

        def scat(j, buf, sem):
            pltpu.async_copy(buf, acc.at[idx_v.at[j]], sem, add=True)

        def wait_s(j, buf, sem):
            pltpu.make_async_copy(buf, acc.at[idx_v.at[j]], sem).wait()

        load_e(0, ebuf0, sem0)
        load_e(1, ebuf1, sem1)
        wait_e(ebuf0, sem0)
        scat(0, ebuf0, sems0)
        count_block(0)
        wait_e(ebuf1, sem1)
        scat(1, ebuf1, sems1)
        count_block(1)
        wait_s(0, ebuf0, sems0)
        load_e(2, ebuf0, sem0)

        def body(t, carry):
            j0 = 2 * t
            j1 = 2 * t + 1
            wait_s(j1 - 2, ebuf1, sems1)
            load_e(j1, ebuf1, sem1)
            wait_e(ebuf0, sem0)
            scat(j0, ebuf0, sems0)
            count_block(j0)
            wait_e(ebuf1, sem1)
            scat(j1, ebuf1, sems1)
            count_block(j1)
            wait_s(j0, ebuf0, sems0)
            load_e(j0 + 2, ebuf0, sem0)
            return carry

        lax.fori_loop(1, (NBLK - 1) // 2, body, 0)
        wait_e(ebuf0, sem0)
        scat(NBLK - 1, ebuf0, sems0)
        count_block(NBLK - 1)
        wait_s(NBLK - 2, ebuf1, sems1)
        wait_s(NBLK - 1, ebuf0, sems0)

        plsc.subcore_barrier()
        pltpu.sync_copy(cnt2d, acc_c.at[iota_v], add=True)
        plsc.subcore_barrier()

        pltpu.sync_copy(acc.at[pl.ds(r0, ROWS_PER_TILE)],
                        sums_out.at[pl.ds(c * N_ACC + r0, ROWS_PER_TILE)])

        @pl.when(s == 0)
        def _():
            pltpu.sync_copy(acc_c, cnt_out.at[pl.ds(c * CROWS, CROWS)])

    return k(recv3d, e)


RBLK = 1024
CR_B = RBLK // H


def _tc_mlp_body(s0_ref, s1_ref, c0_ref, c1_ref, p_ref, b_ref, v_ref,
                 w1a_ref, w1b_ref, b1_ref, w2_ref, b2_ref, out_ref):
    sums = s0_ref[0] + s1_ref[0]
    cimg = c0_ref[0] + c1_ref[0]
    expanded = jnp.dot(p_ref[...], cimg,
                       preferred_element_type=jnp.float32)
    cnt = jnp.sum(expanded * b_ref[...], axis=1, keepdims=True)
    mean = sums / jnp.maximum(cnt, 1.0)
    h = jnp.dot(mean, w1a_ref[...], preferred_element_type=jnp.float32)
    h = h + jnp.dot(v_ref[...], w1b_ref[...], preferred_element_type=jnp.float32)
    h = jnp.maximum(h + b1_ref[...], 0.0)
    o = jnp.dot(h, w2_ref[...], preferred_element_type=jnp.float32)
    out_ref[...] = o + b2_ref[...]


def _tc_mlp(sums3, cnt3, v, W1a, W1b, b1, W2, b2):
    r = np.arange(RBLK)
    P = (r[:, None] >> 7 == np.arange(CR_B)[None, :]).astype(np.float32)
    B = ((r[:, None] & 127) == np.arange(H)[None, :]).astype(np.float32)
    grid = (N_ACC // RBLK,)
    part0 = pl.BlockSpec((1, RBLK, H), lambda i: (0, i, 0))
    part1 = pl.BlockSpec((1, RBLK, H), lambda i: (1, i, 0))
    cim0 = pl.BlockSpec((1, CR_B, H), lambda i: (0, i, 0))
    cim1 = pl.BlockSpec((1, CR_B, H), lambda i: (1, i, 0))
    full = pl.BlockSpec((H, H), lambda i: (0, 0))
    bias = pl.BlockSpec((1, H), lambda i: (0, 0))
    return pl.pallas_call(
        _tc_mlp_body,
        grid=grid,
        in_specs=[part0, part1, cim0, cim1,
                  pl.BlockSpec((RBLK, CR_B), lambda i: (0, 0)),
                  pl.BlockSpec((RBLK, H), lambda i: (0, 0)),
                  pl.BlockSpec((RBLK, H), lambda i: (i, 0)),
                  full, full, bias, full, bias],
        out_specs=pl.BlockSpec((RBLK, H), lambda i: (i, 0)),
        out_shape=jax.ShapeDtypeStruct((N_NODES, H), jnp.float32),
    )(sums3, sums3, cnt3, cnt3, jnp.asarray(P), jnp.asarray(B),
      v, W1a, W1b, b1, W2, b2)


def kernel(v, edge_index, e, W1, b1, W2, b2):
    recv = edge_index[1].astype(jnp.int32).reshape(NW, NBLK, BLK)

    sums, cnt = _sc_fused(recv, e)
    sums = sums.reshape(NC, N_ACC, H)
    cimg = cnt.reshape(NC, CROWS, H)

    return _tc_mlp(sums, cimg, v, W1[:H], W1[H:],
                   b1.reshape(1, H), W2, b2.reshape(1, H))


# speedup vs baseline: 1.3756x; 1.3756x over previous
import functools

import jax
import jax.numpy as jnp
import numpy as np
from jax import lax
from jax.experimental import pallas as pl
from jax.experimental.pallas import tpu as pltpu
from jax.experimental.pallas import tpu_sc as plsc

H = 128
N_NODES = 10000
N_EDGES = 320000

NC = 2
NS = 16
NW = NC * NS

EDGES_PER_TILE = N_EDGES // NW
BLK = 80
NBLK = EDGES_PER_TILE // BLK
N_ACC = 10240
ROWS_PER_TILE = N_ACC // NS
CROWS = N_ACC // H
PROWS = CROWS // 2
ZCH = ROWS_PER_TILE // BLK

MESH = plsc.VectorSubcoreMesh(core_axis_name="c", subcore_axis_name="s")


def _sc_fused(recv3d, e):
    @functools.partial(
        pl.kernel,
        out_type=(
            jax.ShapeDtypeStruct((NC * N_ACC, H), jnp.float32),
            jax.ShapeDtypeStruct((NC * PROWS, H), jnp.int32),
        ),
        mesh=MESH,
        compiler_params=pltpu.CompilerParams(needs_layout_passes=False,
                                             use_tc_tiling_on_sc=False),
        scratch_types=[
            pltpu.VMEM((NBLK, BLK), jnp.int32),
            pltpu.VMEM((BLK, H), jnp.float32),
            pltpu.VMEM((BLK, H), jnp.float32),
            pltpu.VMEM((BLK, H), jnp.float32),
            pltpu.VMEM((PROWS, H), jnp.int32),
            pltpu.VMEM((PROWS,), jnp.int32),
            pltpu.VMEM_SHARED((N_ACC, H), jnp.float32),
            pltpu.VMEM_SHARED((PROWS, H), jnp.int32),
            pltpu.SemaphoreType.DMA,
            pltpu.SemaphoreType.DMA,
            pltpu.SemaphoreType.DMA,
        ],
    )
    def k(recv_hbm, e_hbm,
          sums_out, cnt_out,
          idx_v, ebuf0, ebuf1, ebuf2, cnt2d, iota_v, acc, acc_c,
          sem0, sem1, sem2):
        c = lax.axis_index("c")
        s = lax.axis_index("s")
        w = c * NS + s
        r0 = s * ROWS_PER_TILE
        ebase = w * EDGES_PER_TILE
        ebufs = (ebuf0, ebuf1, ebuf2)
        sems = (sem0, sem1, sem2)

        pltpu.async_copy(recv_hbm.at[w], idx_v, sem0)

        iota_v[pl.ds(0, 16)] = lax.iota(jnp.int32, 16)
        iota_v[pl.ds(16, 16)] = lax.iota(jnp.int32, 16) + 16
        iota_v[pl.ds(24, 16)] = lax.iota(jnp.int32, 16) + 24

        z16i = jnp.zeros((16,), jnp.int32)
        z16f = jnp.zeros((16,), jnp.float32)

        def zrow_c(j, carry):
            for kk in range(H // 16):
                cnt2d[j, pl.ds(kk * 16, 16)] = z16i
            return carry

        lax.fori_loop(0, PROWS, zrow_c, 0)

        def zrow_e(j, carry):
            for kk in range(H // 16):
                ebuf0[j, pl.ds(kk * 16, 16)] = z16f
            return carry

        lax.fori_loop(0, BLK, zrow_e, 0)
        for zz in range(ZCH):
            pltpu.sync_copy(ebuf0, acc.at[pl.ds(r0 + zz * BLK, BLK)])

        @pl.when(s == 0)
        def _():
            pltpu.sync_copy(cnt2d, acc_c)

        pltpu.make_async_copy(recv_hbm.at[w], idx_v, sem0).wait()
        plsc.subcore_barrier()

        one16 = jnp.ones((16,), jnp.int32)

        def count_block(j):
            for kk in range(BLK // 16):
                idx16 = idx_v[j, pl.ds(kk * 16, 16)]
                row = lax.shift_right_logical(idx16, 8)
                col = lax.bitwise_and(lax.shift_right_logical(idx16, 1), 127)
                val = lax.shift_left(
                    one16, lax.shift_left(lax.bitwise_and(idx16, 1), 4))
                plsc.addupdate_scatter(cnt2d, [row, col], val)

        def load_e(j, b):
            pltpu.async_copy(e_hbm.at[pl.ds(ebase + j * BLK, BLK)],
                             ebufs[b], sems[b])

        def substep(j, b):
            load_e(j + 2, (b + 2) % 3)
            count_block(j)
            pltpu.make_async_copy(e_hbm.at[pl.ds(ebase, BLK)],
                                  ebufs[b], sems[b]).wait()
            pltpu.sync_copy(ebufs[b], acc.at[idx_v.at[j]], add=True)

        load_e(0, 0)
        load_e(1, 1)

        def body(t, carry):
            j = 3 * t
            substep(j, 0)
            substep(j + 1, 1)
            substep(j + 2, 2)
            return carry

        lax.fori_loop(0, (NBLK - 2) // 3, body, 0)
        for j, b in ((NBLK - 2, 0), (NBLK - 1, 1)):
            count_block(j)
            pltpu.make_async_copy(e_hbm.at[pl.ds(ebase, BLK)],
                                  ebufs[b], sems[b]).wait()
            pltpu.sync_copy(ebufs[b], acc.at[idx_v.at[j]], add=True)

        plsc.subcore_barrier()
        pltpu.sync_copy(cnt2d, acc_c.at[iota_v], add=True)
        plsc.subcore_barrier()

        pltpu.sync_copy(acc.at[pl.ds(r0, ROWS_PER_TILE)],
                        sums_out.at[pl.ds(c * N_ACC + r0, ROWS_PER_TILE)])

        @pl.when(s == 0)
        def _():
            pltpu.sync_copy(acc_c, cnt_out.at[pl.ds(c * PROWS, PROWS)])

    return k(recv3d, e)


RBLK = 1024
CR_B = RBLK // H


def _tc_mlp_body(s0_ref, s1_ref, c0_ref, c1_ref, p_ref, b_ref, v_ref,
                 w1a_ref, w1b_ref, b1_ref, w2_ref, b2_ref, out_ref):
    sums = s0_ref[0] + s1_ref[0]
    cimg = c0_ref[0] + c1_ref[0]
    expanded = jnp.dot(p_ref[...], cimg,
                       preferred_element_type=jnp.float32)
    cnt = jnp.sum(expanded * b_ref[...], axis=1, keepdims=True)
    mean = sums / jnp.maximum(cnt, 1.0)
    h = jnp.dot(mean, w1a_ref[...], preferred_element_type=jnp.float32)
    h = h + jnp.dot(v_ref[...], w1b_ref[...], preferred_element_type=jnp.float32)
    h = jnp.maximum(h + b1_ref[...], 0.0)
    o = jnp.dot(h, w2_ref[...], preferred_element_type=jnp.float32)
    out_ref[...] = o + b2_ref[...]


def _tc_mlp(sums3, cimg3, v, W1a, W1b, b1, W2, b2):
    r = np.arange(RBLK)
    P = (r[:, None] >> 7 == np.arange(CR_B)[None, :]).astype(np.float32)
    B = ((r[:, None] & 127) == np.arange(H)[None, :]).astype(np.float32)
    grid = (N_ACC // RBLK,)
    part0 = pl.BlockSpec((1, RBLK, H), lambda i: (0, i, 0))
    part1 = pl.BlockSpec((1, RBLK, H), lambda i: (1, i, 0))
    cim0 = pl.BlockSpec((1, CR_B, H), lambda i: (0, i, 0))
    cim1 = pl.BlockSpec((1, CR_B, H), lambda i: (1, i, 0))
    full = pl.BlockSpec((H, H), lambda i: (0, 0))
    bias = pl.BlockSpec((1, H), lambda i: (0, 0))
    return pl.pallas_call(
        _tc_mlp_body,
        grid=grid,
        in_specs=[part0, part1, cim0, cim1,
                  pl.BlockSpec((RBLK, CR_B), lambda i: (0, 0)),
                  pl.BlockSpec((RBLK, H), lambda i: (0, 0)),
                  pl.BlockSpec((RBLK, H), lambda i: (i, 0)),
                  full, full, bias, full, bias],
        out_specs=pl.BlockSpec((RBLK, H), lambda i: (i, 0)),
        out_shape=jax.ShapeDtypeStruct((N_NODES, H), jnp.float32),
    )(sums3, sums3, cimg3, cimg3, jnp.asarray(P), jnp.asarray(B),
      v, W1a, W1b, b1, W2, b2)


def kernel(v, edge_index, e, W1, b1, W2, b2):
    recv = edge_index[1].astype(jnp.int32).reshape(NW, NBLK, BLK)

    sums, cnt32 = _sc_fused(recv, e)
    sums = sums.reshape(NC, N_ACC, H)
    packed = cnt32.reshape(NC, PROWS * H)
    lo = jnp.bitwise_and(packed, 0xFFFF)
    hi = jnp.right_shift(packed, 16)
    cimg = jnp.stack([lo, hi], axis=-1).reshape(NC, CROWS, H).astype(jnp.float32)

    return _tc_mlp(sums, cimg, v, W1[:H], W1[H:],
                   b1.reshape(1, H), W2, b2.reshape(1, H))


# speedup vs baseline: 1.3971x; 1.0156x over previous
import functools

import jax
import jax.numpy as jnp
import numpy as np
from jax import lax
from jax.experimental import pallas as pl
from jax.experimental.pallas import tpu as pltpu
from jax.experimental.pallas import tpu_sc as plsc

H = 128
N_NODES = 10000
N_EDGES = 320000

NC = 2
NS = 16
NW = NC * NS

EDGES_PER_TILE = N_EDGES // NW
BLK = 80
NBLK = EDGES_PER_TILE // BLK
N_ACC = 10240
ROWS_PER_TILE = N_ACC // NS
CROWS = N_ACC // H
PROWS = CROWS // 2
ZCH = ROWS_PER_TILE // BLK

MESH = plsc.VectorSubcoreMesh(core_axis_name="c", subcore_axis_name="s")


def _sc_fused(recv3d, e):
    @functools.partial(
        pl.kernel,
        out_type=(
            jax.ShapeDtypeStruct((NC * N_ACC, H), jnp.float32),
            jax.ShapeDtypeStruct((NC * PROWS, H), jnp.int32),
        ),
        mesh=MESH,
        compiler_params=pltpu.CompilerParams(needs_layout_passes=False,
                                             use_tc_tiling_on_sc=False),
        scratch_types=[
            pltpu.VMEM((NBLK, BLK), jnp.int32),
            pltpu.VMEM((BLK, H), jnp.float32),
            pltpu.VMEM((BLK, H), jnp.float32),
            pltpu.VMEM((BLK, H), jnp.float32),
            pltpu.VMEM((PROWS, H), jnp.int32),
            pltpu.VMEM((PROWS,), jnp.int32),
            pltpu.VMEM_SHARED((N_ACC, H), jnp.float32),
            pltpu.VMEM_SHARED((PROWS, H), jnp.int32),
            pltpu.SemaphoreType.DMA,
            pltpu.SemaphoreType.DMA,
            pltpu.SemaphoreType.DMA,
        ],
    )
    def k(recv_hbm, e_hbm,
          sums_out, cnt_out,
          idx_v, ebuf0, ebuf1, ebuf2, cnt2d, iota_v, acc, acc_c,
          sem0, sem1, sem2):
        c = lax.axis_index("c")
        s = lax.axis_index("s")
        w = c * NS + s
        r0 = s * ROWS_PER_TILE
        ebase = w * EDGES_PER_TILE
        ebufs = (ebuf0, ebuf1, ebuf2)
        sems = (sem0, sem1, sem2)

        pltpu.async_copy(recv_hbm.at[w], idx_v, sem0)

        iota_v[pl.ds(0, 16)] = lax.iota(jnp.int32, 16)
        iota_v[pl.ds(16, 16)] = lax.iota(jnp.int32, 16) + 16
        iota_v[pl.ds(24, 16)] = lax.iota(jnp.int32, 16) + 24

        z16i = jnp.zeros((16,), jnp.int32)
        z16f = jnp.zeros((16,), jnp.float32)

        def zrow_c(j, carry):
            for kk in range(H // 16):
                cnt2d[j, pl.ds(kk * 16, 16)] = z16i
            return carry

        lax.fori_loop(0, PROWS, zrow_c, 0)

        def zrow_e(j, carry):
            for kk in range(H // 16):
                ebuf0[j, pl.ds(kk * 16, 16)] = z16f
            return carry

        lax.fori_loop(0, BLK, zrow_e, 0)
        for zz in range(ZCH):
            pltpu.sync_copy(ebuf0, acc.at[pl.ds(r0 + zz * BLK, BLK)])

        @pl.when(s == 0)
        def _():
            pltpu.sync_copy(cnt2d, acc_c)

        pltpu.make_async_copy(recv_hbm.at[w], idx_v, sem0).wait()
        plsc.subcore_barrier()

        one16 = jnp.ones((16,), jnp.int32)

        def count_block(j):
            for kk in range(BLK // 16):
                idx16 = idx_v[j, pl.ds(kk * 16, 16)]
                row = lax.shift_right_logical(idx16, 8)
                col = lax.bitwise_and(lax.shift_right_logical(idx16, 1), 127)
                val = lax.shift_left(
                    one16, lax.shift_left(lax.bitwise_and(idx16, 1), 4))
                plsc.addupdate_scatter(cnt2d, [row, col], val)

        def load_e(j, b):
            pltpu.async_copy(e_hbm.at[pl.ds(ebase + j * BLK, BLK)],
                             ebufs[b], sems[b])

        def substep(j, b):
            load_e(j + 2, (b + 2) % 3)
            count_block(j)
            pltpu.make_async_copy(e_hbm.at[pl.ds(ebase, BLK)],
                                  ebufs[b], sems[b]).wait()
            pltpu.sync_copy(ebufs[b], acc.at[idx_v.at[j]], add=True)

        load_e(0, 0)
        load_e(1, 1)

        def body(t, carry):
            j = 3 * t
            substep(j, 0)
            substep(j + 1, 1)
            substep(j + 2, 2)
            return carry

        lax.fori_loop(0, (NBLK - 2) // 3, body, 0)
        for j, b in ((NBLK - 2, 0), (NBLK - 1, 1)):
            count_block(j)
            pltpu.make_async_copy(e_hbm.at[pl.ds(ebase, BLK)],
                                  ebufs[b], sems[b]).wait()
            pltpu.sync_copy(ebufs[b], acc.at[idx_v.at[j]], add=True)

        plsc.subcore_barrier()
        pltpu.sync_copy(cnt2d, acc_c.at[iota_v], add=True)
        plsc.subcore_barrier()

        pltpu.sync_copy(acc.at[pl.ds(r0, ROWS_PER_TILE)],
                        sums_out.at[pl.ds(c * N_ACC + r0, ROWS_PER_TILE)])

        @pl.when(s == 0)
        def _():
            pltpu.sync_copy(acc_c, cnt_out.at[pl.ds(c * PROWS, PROWS)])

    return k(recv3d, e)


RBLK = 2048
CR_B = RBLK // H


def _tc_mlp_body(s0_ref, s1_ref, c0_ref, c1_ref, p_ref, b_ref, v_ref,
                 w1a_ref, w1b_ref, b1_ref, w2_ref, b2_ref, out_ref):
    sums = s0_ref[0] + s1_ref[0]
    cimg = c0_ref[0] + c1_ref[0]
    expanded = jnp.dot(p_ref[...], cimg,
                       preferred_element_type=jnp.float32)
    cnt = jnp.sum(expanded * b_ref[...], axis=1, keepdims=True)
    mean = sums / jnp.maximum(cnt, 1.0)
    h = jnp.dot(mean, w1a_ref[...], preferred_element_type=jnp.float32)
    h = h + jnp.dot(v_ref[...], w1b_ref[...], preferred_element_type=jnp.float32)
    h = jnp.maximum(h + b1_ref[...], 0.0)
    o = jnp.dot(h, w2_ref[...], preferred_element_type=jnp.float32)
    out_ref[...] = o + b2_ref[...]


def _tc_mlp(sums3, cimg3, v, W1a, W1b, b1, W2, b2):
    r = np.arange(RBLK)
    P = (r[:, None] >> 7 == np.arange(CR_B)[None, :]).astype(np.float32)
    B = ((r[:, None] & 127) == np.arange(H)[None, :]).astype(np.float32)
    grid = (N_ACC // RBLK,)
    part0 = pl.BlockSpec((1, RBLK, H), lambda i: (0, i, 0))
    part1 = pl.BlockSpec((1, RBLK, H), lambda i: (1, i, 0))
    cim0 = pl.BlockSpec((1, CR_B, H), lambda i: (0, i, 0))
    cim1 = pl.BlockSpec((1, CR_B, H), lambda i: (1, i, 0))
    full = pl.BlockSpec((H, H), lambda i: (0, 0))
    bias = pl.BlockSpec((1, H), lambda i: (0, 0))
    return pl.pallas_call(
        _tc_mlp_body,
        grid=grid,
        in_specs=[part0, part1, cim0, cim1,
                  pl.BlockSpec((RBLK, CR_B), lambda i: (0, 0)),
                  pl.BlockSpec((RBLK, H), lambda i: (0, 0)),
                  pl.BlockSpec((RBLK, H), lambda i: (i, 0)),
                  full, full, bias, full, bias],
        out_specs=pl.BlockSpec((RBLK, H), lambda i: (i, 0)),
        out_shape=jax.ShapeDtypeStruct((N_NODES, H), jnp.float32),
    )(sums3, sums3, cimg3, cimg3, jnp.asarray(P), jnp.asarray(B),
      v, W1a, W1b, b1, W2, b2)


def kernel(v, edge_index, e, W1, b1, W2, b2):
    recv = edge_index[1].astype(jnp.int32).reshape(NW, NBLK, BLK)

    sums, cnt32 = _sc_fused(recv, e)
    sums = sums.reshape(NC, N_ACC, H)
    packed = cnt32.reshape(NC, PROWS * H)
    lo = jnp.bitwise_and(packed, 0xFFFF)
    hi = jnp.right_shift(packed, 16)
    cimg = jnp.stack([lo, hi], axis=-1).reshape(NC, CROWS, H).astype(jnp.float32)

    return _tc_mlp(sums, cimg, v, W1[:H], W1[H:],
                   b1.reshape(1, H), W2, b2.reshape(1, H))


# speedup vs baseline: 1.4334x; 1.0260x over previous
import functools

import jax
import jax.numpy as jnp
import numpy as np
from jax import lax
from jax.experimental import pallas as pl
from jax.experimental.pallas import tpu as pltpu
from jax.experimental.pallas import tpu_sc as plsc

H = 128
N_NODES = 10000
N_EDGES = 320000

NC = 2
NS = 16
NW = NC * NS

EDGES_PER_TILE = N_EDGES // NW
BLK = 80
NBLK = EDGES_PER_TILE // BLK
N_ACC = 10240
ROWS_PER_TILE = N_ACC // NS
CROWS = N_ACC // H
PROWS = CROWS // 2
ZCH = ROWS_PER_TILE // BLK

MESH = plsc.VectorSubcoreMesh(core_axis_name="c", subcore_axis_name="s")


def _sc_fused(recv3d, e):
    @functools.partial(
        pl.kernel,
        out_type=(
            jax.ShapeDtypeStruct((NC * N_ACC, H), jnp.float32),
            jax.ShapeDtypeStruct((NC * PROWS, H), jnp.int32),
        ),
        mesh=MESH,
        compiler_params=pltpu.CompilerParams(needs_layout_passes=False,
                                             use_tc_tiling_on_sc=False),
        scratch_types=[
            pltpu.VMEM((NBLK, BLK), jnp.int32),
            pltpu.VMEM((BLK, H), jnp.float32),
            pltpu.VMEM((BLK, H), jnp.float32),
            pltpu.VMEM((BLK, H), jnp.float32),
            pltpu.VMEM((PROWS, H), jnp.int32),
            pltpu.VMEM((PROWS,), jnp.int32),
            pltpu.VMEM_SHARED((N_ACC, H), jnp.float32),
            pltpu.VMEM_SHARED((PROWS, H), jnp.int32),
            pltpu.SemaphoreType.DMA,
            pltpu.SemaphoreType.DMA,
            pltpu.SemaphoreType.DMA,
        ],
    )
    def k(recv_hbm, e_hbm,
          sums_out, cnt_out,
          idx_v, ebuf0, ebuf1, ebuf2, cnt2d, iota_v, acc, acc_c,
          sem0, sem1, sem2):
        c = lax.axis_index("c")
        s = lax.axis_index("s")
        w = c * NS + s
        r0 = s * ROWS_PER_TILE
        ebase = w * EDGES_PER_TILE
        ebufs = (ebuf0, ebuf1, ebuf2)
        sems = (sem0, sem1, sem2)

        pltpu.async_copy(recv_hbm.at[w], idx_v, sem0)

        iota_v[pl.ds(0, 16)] = lax.iota(jnp.int32, 16)
        iota_v[pl.ds(16, 16)] = lax.iota(jnp.int32, 16) + 16
        iota_v[pl.ds(24, 16)] = lax.iota(jnp.int32, 16) + 24

        z16i = jnp.zeros((16,), jnp.int32)
        z16f = jnp.zeros((16,), jnp.float32)

        def zrow_c(j, carry):
            for kk in range(H // 16):
                cnt2d[j, pl.ds(kk * 16, 16)] = z16i
            return carry

        lax.fori_loop(0, PROWS, zrow_c, 0)

        def zrow_e(j, carry):
            for kk in range(H // 16):
                ebuf0[j, pl.ds(kk * 16, 16)] = z16f
            return carry

        lax.fori_loop(0, BLK, zrow_e, 0)
        for zz in range(ZCH):
            pltpu.sync_copy(ebuf0, acc.at[pl.ds(r0 + zz * BLK, BLK)])

        @pl.when(s == 0)
        def _():
            pltpu.sync_copy(cnt2d, acc_c)

        pltpu.make_async_copy(recv_hbm.at[w], idx_v, sem0).wait()
        plsc.subcore_barrier()

        one16 = jnp.ones((16,), jnp.int32)

        def count_block(j):
            for kk in range(BLK // 16):
                idx16 = idx_v[j, pl.ds(kk * 16, 16)]
                row = lax.shift_right_logical(idx16, 8)
                col = lax.bitwise_and(lax.shift_right_logical(idx16, 1), 127)
                val = lax.shift_left(
                    one16, lax.shift_left(lax.bitwise_and(idx16, 1), 4))
                plsc.addupdate_scatter(cnt2d, [row, col], val)

        def load_e(j, b):
            pltpu.async_copy(e_hbm.at[pl.ds(ebase + j * BLK, BLK)],
                             ebufs[b], sems[b])

        def substep(j, b):
            load_e(j + 2, (b + 2) % 3)
            count_block(j)
            pltpu.make_async_copy(e_hbm.at[pl.ds(ebase, BLK)],
                                  ebufs[b], sems[b]).wait()
            pltpu.sync_copy(ebufs[b], acc.at[idx_v.at[j]], add=True)

        load_e(0, 0)
        load_e(1, 1)

        def body(t, carry):
            j = 3 * t
            substep(j, 0)
            substep(j + 1, 1)
            substep(j + 2, 2)
            return carry

        lax.fori_loop(0, (NBLK - 2) // 3, body, 0)
        for j, b in ((NBLK - 2, 0), (NBLK - 1, 1)):
            count_block(j)
            pltpu.make_async_copy(e_hbm.at[pl.ds(ebase, BLK)],
                                  ebufs[b], sems[b]).wait()
            pltpu.sync_copy(ebufs[b], acc.at[idx_v.at[j]], add=True)

        plsc.subcore_barrier()
        pltpu.sync_copy(cnt2d, acc_c.at[iota_v], add=True)
        plsc.subcore_barrier()

        pltpu.sync_copy(acc.at[pl.ds(r0, ROWS_PER_TILE)],
                        sums_out.at[pl.ds(c * N_ACC + r0, ROWS_PER_TILE)])

        @pl.when(s == 0)
        def _():
            pltpu.sync_copy(acc_c, cnt_out.at[pl.ds(c * PROWS, PROWS)])

    return k(recv3d, e)


RBLK = 2048
CR_B = RBLK // H


def _tc_mlp_body(s0_ref, s1_ref, c0_ref, c1_ref, p_ref, b_ref, m_ref, v_ref,
                 w1a_ref, w1b_ref, b1_ref, w2_ref, b2_ref, out_ref):
    sums = s0_ref[0] + s1_ref[0]
    pk = c0_ref[0] + c1_ref[0]
    lo = jnp.bitwise_and(pk, 0xFFFF).astype(jnp.float32)
    hi = jnp.right_shift(pk, 16).astype(jnp.float32)
    eL = jnp.dot(p_ref[...], lo, preferred_element_type=jnp.float32)
    eH = jnp.dot(p_ref[...], hi, preferred_element_type=jnp.float32)
    mixed = (eL * m_ref[...] + eH * (1.0 - m_ref[...])) * b_ref[...]
    cnt = jnp.sum(mixed, axis=1, keepdims=True)
    mean = sums / jnp.maximum(cnt, 1.0)
    h = jnp.dot(mean, w1a_ref[...], preferred_element_type=jnp.float32)
    h = h + jnp.dot(v_ref[...], w1b_ref[...], preferred_element_type=jnp.float32)
    h = jnp.maximum(h + b1_ref[...], 0.0)
    o = jnp.dot(h, w2_ref[...], preferred_element_type=jnp.float32)
    out_ref[...] = o + b2_ref[...]


PB = RBLK // 256


def _tc_mlp(sums3, cimg3, v, W1a, W1b, b1, W2, b2):
    r = np.arange(RBLK)
    P = (r[:, None] >> 8 == np.arange(PB)[None, :]).astype(np.float32)
    B = (((r[:, None] >> 1) & 127) == np.arange(H)[None, :]).astype(np.float32)
    M = ((r[:, None] & 1) == 0).astype(np.float32)
    grid = (N_ACC // RBLK,)
    part0 = pl.BlockSpec((1, RBLK, H), lambda i: (0, i, 0))
    part1 = pl.BlockSpec((1, RBLK, H), lambda i: (1, i, 0))
    cim0 = pl.BlockSpec((1, PB, H), lambda i: (0, i, 0))
    cim1 = pl.BlockSpec((1, PB, H), lambda i: (1, i, 0))
    full = pl.BlockSpec((H, H), lambda i: (0, 0))
    bias = pl.BlockSpec((1, H), lambda i: (0, 0))
    return pl.pallas_call(
        _tc_mlp_body,
        grid=grid,
        in_specs=[part0, part1, cim0, cim1,
                  pl.BlockSpec((RBLK, PB), lambda i: (0, 0)),
                  pl.BlockSpec((RBLK, H), lambda i: (0, 0)),
                  pl.BlockSpec((RBLK, 1), lambda i: (0, 0)),
                  pl.BlockSpec((RBLK, H), lambda i: (i, 0)),
                  full, full, bias, full, bias],
        out_specs=pl.BlockSpec((RBLK, H), lambda i: (i, 0)),
        out_shape=jax.ShapeDtypeStruct((N_NODES, H), jnp.float32),
    )(sums3, sums3, cimg3, cimg3, jnp.asarray(P), jnp.asarray(B),
      jnp.asarray(M), v, W1a, W1b, b1, W2, b2)


def kernel(v, edge_index, e, W1, b1, W2, b2):
    recv = edge_index[1].astype(jnp.int32).reshape(NW, NBLK, BLK)

    sums, cnt32 = _sc_fused(recv, e)
    sums = sums.reshape(NC, N_ACC, H)
    cimg = cnt32.reshape(NC, PROWS, H)

    return _tc_mlp(sums, cimg, v, W1[:H], W1[H:],
                   b1.reshape(1, H), W2, b2.reshape(1, H))
